# Initial kernel scaffold; baseline (speedup 1.0000x reference)
import jax
import jax.numpy as jnp
from jax.experimental import pallas as pl


def kernel(x, edge_index, edge_attr, W1, b1, root1, We1, be1, Wmu, bmu, rootmu, Wemu, bemu, Wls, bls, rootls, Wels, bels):
    raise NotImplementedError("write your pallas kernel here")


# speedup vs baseline: 6.6229x; 6.6229x over previous
import functools
import jax
import jax.numpy as jnp
from jax import lax
from jax.experimental import pallas as pl
from jax.experimental.pallas import tpu as pltpu
from jax.experimental.pallas import tpu_sc as plsc

F32 = jnp.float32
CHUNK = 128
NUM_WORKERS = 32


def _pads(n_nodes, n_edges):
    npad = ((n_nodes + 1 + 127) // 128) * 128
    epad = ((n_edges + NUM_WORKERS * CHUNK - 1) // (NUM_WORKERS * CHUNK)) * (
        NUM_WORKERS * CHUNK)
    return npad, epad


def _zero_rows(buf, ncols16):
    def body(i, _):
        for k in range(ncols16):
            buf[i, pl.ds(k * 16, 16)] = jnp.zeros((16,), F32)
        return 0
    lax.fori_loop(0, CHUNK, body, 0)


def _zero_acc_slice(zbuf, acc_sh, sid, rows_per_tile):
    base = sid * rows_per_tile
    full, rem = rows_per_tile // CHUNK, rows_per_tile % CHUNK
    for r in range(full):
        pltpu.sync_copy(zbuf, acc_sh.at[pl.ds(base + r * CHUNK, CHUNK)])
    if rem:
        pltpu.sync_copy(zbuf.at[pl.ds(0, rem)],
                        acc_sh.at[pl.ds(base + full * CHUNK, rem)])


def _deg_pass(sidx2d, npad):
    nchunk = sidx2d.shape[0]
    per_tile = nchunk // NUM_WORKERS
    rows_per_tile = npad // 16
    mesh = plsc.VectorSubcoreMesh(core_axis_name="c", subcore_axis_name="s")

    @functools.partial(
        pl.kernel,
        out_type=jax.ShapeDtypeStruct((2, npad, 16), F32),
        mesh=mesh,
        scratch_types=[
            pltpu.VMEM((CHUNK,), jnp.int32),
            pltpu.VMEM((CHUNK, 16), F32),
            pltpu.VMEM_SHARED((npad, 16), F32),
        ],
    )
    def k(sidx_hbm, out_hbm, idx_v, val_v, acc_sh):
        cid = lax.axis_index("c")
        sid = lax.axis_index("s")
        wid = cid * 16 + sid
        _zero_rows(val_v, 1)
        _zero_acc_slice(val_v, acc_sh, sid, rows_per_tile)
        plsc.subcore_barrier()

        def fill_ones(i, _):
            val_v[i] = jnp.ones((16,), F32)
            return 0
        lax.fori_loop(0, CHUNK, fill_ones, 0)

        def body(t, _):
            chunk = wid * per_tile + t
            pltpu.sync_copy(sidx_hbm.at[chunk], idx_v)
            pltpu.sync_copy(val_v, acc_sh.at[idx_v], add=True)
            return 0
        lax.fori_loop(0, per_tile, body, 0)
        plsc.subcore_barrier()
        pltpu.sync_copy(acc_sh.at[pl.ds(sid * rows_per_tile, rows_per_tile)],
                        out_hbm.at[cid].at[pl.ds(sid * rows_per_tile, rows_per_tile)])

    return k(sidx2d)


def _edge_pass(h_tbl, ee_tbl, dinv16, sidx2d, didx2d):
    npad, d = h_tbl.shape
    nchunk = sidx2d.shape[0]
    per_tile = nchunk // NUM_WORKERS
    rows_per_tile = npad // 16
    nd16 = d // 16
    mesh = plsc.VectorSubcoreMesh(core_axis_name="c", subcore_axis_name="s")

    @functools.partial(
        pl.kernel,
        out_type=jax.ShapeDtypeStruct((2, npad, d), F32),
        mesh=mesh,
        scratch_types=[
            pltpu.VMEM((CHUNK,), jnp.int32),
            pltpu.VMEM((CHUNK,), jnp.int32),
            pltpu.VMEM((CHUNK, d), F32),
            pltpu.VMEM((CHUNK, d), F32),
            pltpu.VMEM((CHUNK, 128), F32),
            pltpu.VMEM_SHARED((npad, d), F32),
        ],
    )
    def k(h_hbm, ee_hbm, dinv_hbm, sidx_hbm, didx_hbm, out_hbm,
          sidx_v, didx_v, hbuf, eebuf, svec, acc_sh):
        cid = lax.axis_index("c")
        sid = lax.axis_index("s")
        wid = cid * 16 + sid
        _zero_rows(hbuf, nd16)
        _zero_acc_slice(hbuf, acc_sh, sid, rows_per_tile)
        plsc.subcore_barrier()

        def body(t, _):
            chunk = wid * per_tile + t
            pltpu.sync_copy(sidx_hbm.at[chunk], sidx_v)
            pltpu.sync_copy(didx_hbm.at[chunk], didx_v)
            pltpu.sync_copy(h_hbm.at[sidx_v], hbuf)
            pltpu.sync_copy(dinv_hbm.at[sidx_v], svec)
            pltpu.sync_copy(ee_hbm.at[pl.ds(chunk * CHUNK, CHUNK)], eebuf)

            def edge(e, _):
                s = svec[e, 0:16]
                for kk in range(nd16):
                    v = hbuf[e, pl.ds(kk * 16, 16)] + eebuf[e, pl.ds(kk * 16, 16)]
                    hbuf[e, pl.ds(kk * 16, 16)] = jnp.maximum(v, 0.0) * s
                return 0
            lax.fori_loop(0, CHUNK, edge, 0)
            pltpu.sync_copy(hbuf, acc_sh.at[didx_v], add=True)
            return 0
        lax.fori_loop(0, per_tile, body, 0)
        plsc.subcore_barrier()
        pltpu.sync_copy(acc_sh.at[pl.ds(sid * rows_per_tile, rows_per_tile)],
                        out_hbm.at[cid].at[pl.ds(sid * rows_per_tile, rows_per_tile)])

    return k(h_tbl, ee_tbl, dinv16, sidx2d, didx2d)


def _dinv_from_deg(degp):
    npad = degp.shape[1]

    def body(p_ref, o_ref):
        deg = p_ref[0, :, 0:1] + p_ref[1, :, 0:1] + 1.0
        o_ref[:] = jnp.broadcast_to(lax.rsqrt(deg), (npad, 128))
    return pl.pallas_call(
        body,
        out_shape=jax.ShapeDtypeStruct((npad, 128), F32),
    )(degp)


def _node_matmul(xp, wt, b):
    def body(x_ref, w_ref, b_ref, o_ref):
        o_ref[:] = jnp.dot(x_ref[:], w_ref[:],
                           preferred_element_type=F32) + b_ref[:]
    return pl.pallas_call(
        body,
        out_shape=jax.ShapeDtypeStruct((xp.shape[0], wt.shape[1]), F32),
    )(xp, wt, b)


def _edge_matmul(ea8, wt8, b):
    epad = ea8.shape[0]
    dout = wt8.shape[1]
    blk = 4096
    grid = epad // blk

    def body(a_ref, w_ref, b_ref, o_ref):
        o_ref[:] = jnp.dot(a_ref[:], w_ref[:],
                           preferred_element_type=F32) + b_ref[:]
    return pl.pallas_call(
        body,
        grid=(grid,),
        in_specs=[
            pl.BlockSpec((blk, 8), lambda i: (i, 0)),
            pl.BlockSpec((8, dout), lambda i: (0, 0)),
            pl.BlockSpec((1, dout), lambda i: (0, 0)),
        ],
        out_specs=pl.BlockSpec((blk, dout), lambda i: (i, 0)),
        out_shape=jax.ShapeDtypeStruct((epad, dout), F32),
    )(ea8, wt8, b)


def _combine_project(parts, hpre, dinv16, root, wt, b):
    def body(p_ref, h_ref, d_ref, r_ref, w_ref, b_ref, o_ref):
        dinv = d_ref[:, 0:1]
        invdeg = dinv * dinv
        agg = (p_ref[0] + p_ref[1]) * dinv
        self_t = jnp.maximum(h_ref[:] + r_ref[:], 0.0) * invdeg
        h = jnp.maximum(agg + self_t, 0.0)
        o_ref[:] = jnp.dot(h, w_ref[:], preferred_element_type=F32) + b_ref[:]
    return pl.pallas_call(
        body,
        out_shape=jax.ShapeDtypeStruct((hpre.shape[0], wt.shape[1]), F32),
    )(parts, hpre, dinv16, root, wt, b)


def _combine_final(parts, hpre, dinv16, root):
    def body(p_ref, h_ref, d_ref, r_ref, o_ref):
        dinv = d_ref[:, 0:1]
        invdeg = dinv * dinv
        agg = (p_ref[0] + p_ref[1]) * dinv
        o_ref[:] = agg + jnp.maximum(h_ref[:] + r_ref[:], 0.0) * invdeg
    return pl.pallas_call(
        body,
        out_shape=jax.ShapeDtypeStruct(hpre.shape, F32),
    )(parts, hpre, dinv16, root)


def kernel(x, edge_index, edge_attr,
           W1, b1, root1, We1, be1,
           Wmu, bmu, rootmu, Wemu, bemu,
           Wls, bls, rootls, Wels, bels):
    n, d_in = x.shape
    e = edge_index.shape[1]
    d_edge = edge_attr.shape[1]
    npad, epad = _pads(n, e)

    xp = jnp.pad(x, ((0, npad - n), (0, 0)))
    pad_cnt = epad - e
    dump = n + (jnp.arange(pad_cnt, dtype=jnp.int32) % (npad - n))
    sidx2d = jnp.concatenate([edge_index[0], dump]).reshape(-1, CHUNK)
    didx2d = jnp.concatenate([edge_index[1], dump]).reshape(-1, CHUNK)
    ea8 = jnp.pad(edge_attr, ((0, pad_cnt), (0, 8 - d_edge)))

    w1t = W1.T
    wcat_t = jnp.concatenate([Wmu.T, Wls.T], axis=1)
    bcat = jnp.concatenate([bmu, bls]).reshape(1, -1)
    rootcat = jnp.concatenate([rootmu, rootls], axis=1)
    we1t8 = jnp.pad(We1.T, ((0, 8 - d_edge), (0, 0)))
    wecat_t8 = jnp.pad(jnp.concatenate([Wemu.T, Wels.T], axis=1),
                       ((0, 8 - d_edge), (0, 0)))
    becat = jnp.concatenate([bemu, bels]).reshape(1, -1)

    degp = _deg_pass(sidx2d, npad)
    dinv16 = _dinv_from_deg(degp)

    h1pre = _node_matmul(xp, w1t, b1.reshape(1, -1))
    ee1 = _edge_matmul(ea8, we1t8, be1.reshape(1, -1))
    p1 = _edge_pass(h1pre, ee1, dinv16, sidx2d, didx2d)
    hcat = _combine_project(p1, h1pre, dinv16, root1, wcat_t, bcat)

    eecat = _edge_matmul(ea8, wecat_t8, becat)
    pcat = _edge_pass(hcat, eecat, dinv16, sidx2d, didx2d)
    outcat = _combine_final(pcat, hcat, dinv16, rootcat)

    d_out = Wmu.shape[0]
    return (outcat[:n, :d_out], outcat[:n, d_out:2 * d_out])
